# baseline (speedup 1.0000x reference)
import functools
import math

import jax
import jax.numpy as jnp
from jax.experimental import pallas as pl
from jax.experimental.pallas import tpu as pltpu

TOPK = 16
NEG = -1e30
RB = 64


def _attn_block_kernel(q_ref, k_ref, v_ref, o_ref, *, sq, L):
    sb = pl.program_id(1)
    q = q_ref[0]
    k = k_ref[0]
    v = v_ref[0]

    s = jax.lax.dot_general(
        q, k, (((1,), (1,)), ((), ())),
        preferred_element_type=jnp.float32)

    row = jax.lax.broadcasted_iota(jnp.int32, (sq, L), 0) + sb * sq
    col = jax.lax.broadcasted_iota(jnp.int32, (sq, L), 1)
    s = jnp.where(col <= row, s, NEG)

    G = L // 4
    t_parts = []
    m_parts = []
    for r in range(sq // RB):
        sr = s[r * RB:(r + 1) * RB, :]
        a, b, c, d = (sr[:, 0:G], sr[:, G:2 * G],
                      sr[:, 2 * G:3 * G], sr[:, 3 * G:4 * G])
        hi1, lo1 = jnp.maximum(a, b), jnp.minimum(a, b)
        hi2, lo2 = jnp.maximum(c, d), jnp.minimum(c, d)
        w = jnp.maximum(hi1, hi2)
        th = jnp.minimum(hi1, hi2)
        tl = jnp.maximum(lo1, lo2)
        s4 = jnp.minimum(lo1, lo2)
        s2 = jnp.maximum(th, tl)
        s3 = jnp.minimum(th, tl)

        m = None
        t = None
        for p in range(TOPK):
            t = jnp.max(w, axis=1, keepdims=True)
            if p == 0:
                m = t
            if p < TOPK - 1:
                hit = w == t
                w = jnp.where(hit, s2, w)
                s2 = jnp.where(hit, s3, s2)
                s3 = jnp.where(hit, s4, s3)
                s4 = jnp.where(hit, NEG, s4)
        t_parts.append(t)
        m_parts.append(m)

    t = jnp.concatenate(t_parts, axis=0)
    m = jnp.concatenate(m_parts, axis=0)

    p_num = jnp.where(s >= t, jnp.exp(s - m), 0.0)
    z = jnp.sum(p_num, axis=1, keepdims=True)
    o = jax.lax.dot_general(
        p_num, v, (((1,), (0,)), ((), ())),
        preferred_element_type=jnp.float32)
    o_ref[0] = o / z


@jax.jit
def kernel(q, compressed_k, compressed_v):
    B, H, S, D = q.shape
    L = compressed_k.shape[2]
    q3 = q.reshape(H, S, D) * (1.0 / math.sqrt(D))
    k3 = compressed_k.reshape(H, L, D)
    v3 = compressed_v.reshape(H, L, D)

    SQ = 512
    grid = (H, S // SQ)
    out = pl.pallas_call(
        functools.partial(_attn_block_kernel, sq=SQ, L=L),
        grid=grid,
        in_specs=[
            pl.BlockSpec((1, SQ, D), lambda h, sb: (h, sb, 0)),
            pl.BlockSpec((1, L, D), lambda h, sb: (h, 0, 0)),
            pl.BlockSpec((1, L, D), lambda h, sb: (h, 0, 0)),
        ],
        out_specs=pl.BlockSpec((1, SQ, D), lambda h, sb: (h, sb, 0)),
        out_shape=jax.ShapeDtypeStruct((H, S, D), jnp.float32),
        compiler_params=pltpu.CompilerParams(
            dimension_semantics=("parallel", "parallel")),
    )(q3, k3, v3)
    return out.reshape(B, H, S, D)


# speedup vs baseline: 1.1889x; 1.1889x over previous
import functools
import math

import jax
import jax.numpy as jnp
from jax.experimental import pallas as pl
from jax.experimental.pallas import tpu as pltpu

TOPK = 16
NEG = -1e30


def _attn_block_kernel(q_ref, k_ref, v_ref, o_ref, *, sq, L, scale):
    sb = pl.program_id(1)
    q = q_ref[0] * scale
    k = k_ref[0]
    v = v_ref[0]

    s = jax.lax.dot_general(
        q, k, (((1,), (1,)), ((), ())),
        preferred_element_type=jnp.float32)

    row = jax.lax.broadcasted_iota(jnp.int32, (sq, L), 0) + sb * sq
    col = jax.lax.broadcasted_iota(jnp.int32, (sq, L), 1)
    s = jnp.where(col <= row, s, NEG)

    a, b, c, d = (s[:, 0:128], s[:, 128:256], s[:, 256:384], s[:, 384:512])
    hi1, lo1 = jnp.maximum(a, b), jnp.minimum(a, b)
    hi2, lo2 = jnp.maximum(c, d), jnp.minimum(c, d)
    s1 = jnp.maximum(hi1, hi2)
    th = jnp.minimum(hi1, hi2)
    tl = jnp.maximum(lo1, lo2)
    s4 = jnp.minimum(lo1, lo2)
    s2 = jnp.maximum(th, tl)
    s3 = jnp.minimum(th, tl)

    w = s1
    m = None
    t = None
    for p in range(TOPK):
        t = jnp.max(w, axis=1, keepdims=True)
        if p == 0:
            m = t
        if p < TOPK - 1:
            hit = w == t
            w = jnp.where(hit, s2, w)
            s2 = jnp.where(hit, s3, s2)
            s3 = jnp.where(hit, s4, s3)
            s4 = jnp.where(hit, NEG, s4)

    p_num = jnp.where(s >= t, jnp.exp(s - m), 0.0)
    z = jnp.sum(p_num, axis=1, keepdims=True)
    o = jax.lax.dot_general(
        p_num, v, (((1,), (0,)), ((), ())),
        preferred_element_type=jnp.float32)
    o_ref[0] = o / z


@jax.jit
def kernel(q, compressed_k, compressed_v):
    B, H, S, D = q.shape
    L = compressed_k.shape[2]
    q3 = q.reshape(H, S, D)
    k3 = compressed_k.reshape(H, L, D)
    v3 = compressed_v.reshape(H, L, D)

    SQ = 512
    grid = (H, S // SQ)
    out = pl.pallas_call(
        functools.partial(_attn_block_kernel, sq=SQ, L=L,
                          scale=1.0 / math.sqrt(D)),
        grid=grid,
        in_specs=[
            pl.BlockSpec((1, SQ, D), lambda h, sb: (h, sb, 0)),
            pl.BlockSpec((1, L, D), lambda h, sb: (h, 0, 0)),
            pl.BlockSpec((1, L, D), lambda h, sb: (h, 0, 0)),
        ],
        out_specs=pl.BlockSpec((1, SQ, D), lambda h, sb: (h, sb, 0)),
        out_shape=jax.ShapeDtypeStruct((H, S, D), jnp.float32),
        compiler_params=pltpu.CompilerParams(
            dimension_semantics=("parallel", "parallel")),
    )(q3, k3, v3)
    return out.reshape(B, H, S, D)


# speedup vs baseline: 1.2588x; 1.0588x over previous
import functools
import math

import jax
import jax.numpy as jnp
from jax.experimental import pallas as pl
from jax.experimental.pallas import tpu as pltpu

TOPK = 16
NEG = -1e30


def _attn_block_kernel(q_ref, k_ref, v_ref, o_ref, *, sq, L, scale):
    sb = pl.program_id(1)
    q = q_ref[0, 0] * scale
    k = k_ref[0, 0]
    v = v_ref[0, 0]

    s = jax.lax.dot_general(
        q, k, (((1,), (1,)), ((), ())),
        preferred_element_type=jnp.float32)

    row = jax.lax.broadcasted_iota(jnp.int32, (sq, L), 0) + sb * sq
    col = jax.lax.broadcasted_iota(jnp.int32, (sq, L), 1)
    s = jnp.where(col <= row, s, NEG)

    a, b, c, d = (s[:, 0:128], s[:, 128:256], s[:, 256:384], s[:, 384:512])
    hi1, lo1 = jnp.maximum(a, b), jnp.minimum(a, b)
    hi2, lo2 = jnp.maximum(c, d), jnp.minimum(c, d)
    s1 = jnp.maximum(hi1, hi2)
    th = jnp.minimum(hi1, hi2)
    tl = jnp.maximum(lo1, lo2)
    s4 = jnp.minimum(lo1, lo2)
    s2 = jnp.maximum(th, tl)
    s3 = jnp.minimum(th, tl)

    w = s1
    m = None
    t = None
    for p in range(TOPK):
        t = jnp.max(w, axis=1, keepdims=True)
        if p == 0:
            m = t
        if p < TOPK - 1:
            hit = w == t
            w = jnp.where(hit, s2, w)
            s2 = jnp.where(hit, s3, s2)
            s3 = jnp.where(hit, s4, s3)
            s4 = jnp.where(hit, NEG, s4)

    p_num = jnp.where(s >= t, jnp.exp(s - m), 0.0)
    z = jnp.sum(p_num, axis=1, keepdims=True)
    o = jax.lax.dot_general(
        p_num, v, (((1,), (0,)), ((), ())),
        preferred_element_type=jnp.float32)
    o_ref[0, 0] = o / z


@jax.jit
def kernel(q, compressed_k, compressed_v):
    B, H, S, D = q.shape
    L = compressed_k.shape[2]
    SQ = 512
    grid = (H, S // SQ)
    return pl.pallas_call(
        functools.partial(_attn_block_kernel, sq=SQ, L=L,
                          scale=1.0 / math.sqrt(D)),
        grid=grid,
        in_specs=[
            pl.BlockSpec((1, 1, SQ, D), lambda h, sb: (0, h, sb, 0)),
            pl.BlockSpec((1, 1, L, D), lambda h, sb: (0, h, 0, 0)),
            pl.BlockSpec((1, 1, L, D), lambda h, sb: (0, h, 0, 0)),
        ],
        out_specs=pl.BlockSpec((1, 1, SQ, D), lambda h, sb: (0, h, sb, 0)),
        out_shape=jax.ShapeDtypeStruct((B, H, S, D), jnp.float32),
        compiler_params=pltpu.CompilerParams(
            dimension_semantics=("parallel", "parallel")),
    )(q, compressed_k, compressed_v)
